# baseline (speedup 1.0000x reference)
import functools

import jax
import jax.numpy as jnp
from jax import lax
from jax.experimental import pallas as pl
from jax.experimental.pallas import tpu as pltpu
from jax.experimental.pallas import tpu_sc as plsc

N = 10000
E = 320000
F = 128
NP = 10240
NC = 2
NS = 16
NW = NC * NS
EPW = E // NW
EB = 80
NB = EPW // EB
GR = 3
NGF = NB // GR
TB = NB % GR
GROWS = GR * EB
SEB = 128
SNB = EPW // SEB
STL = EPW - SNB * SEB
SGR = 2
SNGF = SNB // SGR
SGROWS = SGR * SEB
RPS = NP // NS
DGR = 5

_mesh = plsc.VectorSubcoreMesh(core_axis_name="c", subcore_axis_name="s")


@functools.partial(
    pl.kernel,
    out_type=jax.ShapeDtypeStruct((NC, NP), jnp.float32),
    mesh=_mesh,
    scratch_types=(
        [
            pltpu.VMEM_SHARED((NP,), jnp.float32),
            pltpu.VMEM((RPS,), jnp.float32),
            pltpu.VMEM((EB,), jnp.float32),
        ]
        + [pltpu.VMEM((EB,), jnp.int32) for _ in range(DGR)]
        + [pltpu.SemaphoreType.DMA for _ in range(2 * DGR)]
    ),
)
def _deg_kernel(row_hbm, degp_hbm, acc, zbuf, onesv, *rest):
  idxv = rest[:DGR]
  isem = rest[DGR:2 * DGR]
  ssem = rest[2 * DGR:3 * DGR]
  c = lax.axis_index("c")
  s = lax.axis_index("s")
  wid = c * NS + s
  base = wid * EPW

  z = jnp.zeros((16,), dtype=jnp.float32)
  for i in range(RPS // 16):
    zbuf[pl.ds(16 * i, 16)] = z
  o = jnp.ones((16,), dtype=jnp.float32)
  for i in range(EB // 16):
    onesv[pl.ds(16 * i, 16)] = o
  pltpu.sync_copy(zbuf, acc.at[pl.ds(RPS * s, RPS)])
  plsc.subcore_barrier()

  def dgroup(g, drain):
    il = []
    for r in range(DGR):
      if drain:
        pltpu.make_async_copy(row_hbm.at[pl.ds(0, EB)], idxv[r],
                              ssem[r]).wait()
      off = base + EB * (g * DGR + r)
      il.append(pltpu.async_copy(row_hbm.at[pl.ds(off, EB)], idxv[r],
                                 isem[r]))
    for r in range(DGR):
      il[r].wait()
      pltpu.async_copy(onesv, acc.at[idxv[r]], ssem[r], add=True)

  dgroup(0, False)

  def body(g, carry):
    dgroup(g, True)
    return carry

  lax.fori_loop(1, NB // DGR, body, 0)
  for r in range(DGR):
    pltpu.make_async_copy(row_hbm.at[pl.ds(0, EB)], idxv[r], ssem[r]).wait()
  plsc.subcore_barrier()
  pltpu.sync_copy(acc.at[pl.ds(RPS * s, RPS)], zbuf)
  pltpu.sync_copy(zbuf, degp_hbm.at[c, pl.ds(RPS * s, RPS)])


@functools.partial(
    pl.kernel,
    out_type=jax.ShapeDtypeStruct((NC, NP, F), jnp.float32),
    mesh=_mesh,
    scratch_types=(
        [
            pltpu.VMEM_SHARED((NP, F), jnp.float32),
            pltpu.VMEM((SGROWS, F), jnp.float32),
        ]
        + [pltpu.VMEM((EPW,), jnp.int32)]
        + [pltpu.VMEM((40, F), jnp.float32)]
        + [pltpu.VMEM((16,), jnp.int32)]
        + [pltpu.VMEM((SEB,), jnp.int32) for _ in range(SGR)]
        + [pltpu.SemaphoreType.DMA for _ in range(3 * SGR)]
    ),
)
def _scatter_kernel(tab_hbm, row_hbm, col_hbm, out_hbm, acc, rows_v, *rest):
  ridx = rest[0]
  zbuf = rest[1]
  tidx = rest[2]
  cidx = rest[3:3 + SGR]
  sems = rest[3 + SGR:]
  cisem = sems[:SGR]
  gsem = sems[SGR:2 * SGR]
  ssem = sems[2 * SGR:3 * SGR]
  c = lax.axis_index("c")
  s = lax.axis_index("s")
  wid = c * NS + s
  base = wid * EPW

  pltpu.async_copy(row_hbm.at[pl.ds(base, EPW)], ridx, cisem[0]).wait()
  g0gd, g0cl = [], []
  for r in range(SGR):
    g0gd.append(pltpu.async_copy(tab_hbm.at[ridx.at[pl.ds(SEB * r, SEB)]],
                                 rows_v.at[pl.ds(SEB * r, SEB)], gsem[r]))
    g0cl.append(pltpu.async_copy(col_hbm.at[pl.ds(base + SEB * r, SEB)],
                                 cidx[r], cisem[r]))

  def zrow(r, carry):
    z = jnp.zeros((16,), dtype=jnp.float32)
    for k in range(F // 16):
      zbuf[r, pl.ds(16 * k, 16)] = z
    return carry

  lax.fori_loop(0, 40, zrow, 0)
  zd = []
  for j in range(RPS // 40):
    zd.append(pltpu.async_copy(zbuf, acc.at[pl.ds(RPS * s + 40 * j, 40)],
                               ssem[j % SGR]))
  for d in zd:
    d.wait()
  plsc.subcore_barrier()
  for r in range(SGR):
    g0gd[r].wait()
    g0cl[r].wait()
    pltpu.async_copy(rows_v.at[pl.ds(SEB * r, SEB)], acc.at[cidx[r]],
                     ssem[r], add=True)

  def drain_scatter(r):
    pltpu.make_async_copy(tab_hbm.at[pl.ds(0, SEB)],
                          rows_v.at[pl.ds(SEB * r, SEB)], ssem[r]).wait()

  def group(g, carry):
    cl, gd = [], []
    for r in range(SGR):
      drain_scatter(r)
      off = base + SEB * (g * SGR + r)
      cl.append(pltpu.async_copy(col_hbm.at[pl.ds(off, SEB)], cidx[r],
                                 cisem[r]))
      gd.append(
          pltpu.async_copy(
              tab_hbm.at[ridx.at[pl.ds(SEB * (g * SGR + r), SEB)]],
              rows_v.at[pl.ds(SEB * r, SEB)], gsem[r]))
    for r in range(SGR):
      gd[r].wait()
      cl[r].wait()
      pltpu.async_copy(rows_v.at[pl.ds(SEB * r, SEB)], acc.at[cidx[r]],
                       ssem[r], add=True)
    return carry

  lax.fori_loop(1, SNGF, group, 0)
  for r in range(SGR):
    drain_scatter(r)
  if STL:
    off = base + SNB * SEB
    pltpu.sync_copy(col_hbm.at[pl.ds(off, STL)], tidx)
    pltpu.async_copy(tab_hbm.at[ridx.at[pl.ds(SNB * SEB, STL)]],
                     rows_v.at[pl.ds(0, STL)], gsem[0]).wait()
    pltpu.sync_copy(rows_v.at[pl.ds(0, STL)], acc.at[tidx], add=True)
  plsc.subcore_barrier()
  dd = []
  for j in range(RPS // 160):
    off = RPS * s + 160 * j
    dd.append(pltpu.async_copy(acc.at[pl.ds(off, 160)],
                               out_hbm.at[c, pl.ds(off, 160)],
                               gsem[j % SGR]))
  for d in dd:
    d.wait()


def _mlp_body(x_ref, w1_ref, b1_ref, w2_ref, b2_ref, degp_ref, h_ref,
              g0_ref):
  i = pl.program_id(0)
  h1 = lax.dot_general(x_ref[...], w1_ref[...], (((1,), (1,)), ((), ())),
                       preferred_element_type=jnp.float32)
  h1 = jnp.maximum(h1 + b1_ref[...], 0.0)
  h = lax.dot_general(h1, w2_ref[...], (((1,), (1,)), ((), ())),
                      preferred_element_type=jnp.float32)
  h = h + b2_ref[...]
  h_ref[...] = h
  dp = degp_ref[:, pl.ds(i * _BLK, _BLK)]
  deg = jnp.sum(dp, axis=0)
  dis = jnp.where(deg > 0, lax.rsqrt(deg), 0.0)
  g0_ref[...] = h * dis[:, None]


def _scale_body(p_ref, degp_ref, g1_ref):
  i = pl.program_id(0)
  ps = p_ref[0] + p_ref[1]
  dp = degp_ref[:, pl.ds(i * _BLK, _BLK)]
  deg = jnp.sum(dp, axis=0)
  dis2 = jnp.where(deg > 0, 1.0 / deg, 0.0)
  g1_ref[...] = ps * dis2[:, None]


def _final_body(q_ref, degp_ref, h_ref, o_ref):
  i = pl.program_id(0)
  qs = q_ref[0] + q_ref[1]
  dp = degp_ref[:, pl.ds(i * _BLK, _BLK)]
  deg = jnp.sum(dp, axis=0)
  dis = jnp.where(deg > 0, lax.rsqrt(deg), 0.0)
  o_ref[...] = 0.1 * h_ref[...] + 0.081 * (qs * dis[:, None])


_BLK = 2048
_GRID = NP // _BLK
_full_deg_spec = pl.BlockSpec((NC, NP), lambda i: (0, 0))
_row_spec = pl.BlockSpec((_BLK, F), lambda i: (i, 0))
_w_spec = pl.BlockSpec((F, F), lambda i: (0, 0))
_b_spec = pl.BlockSpec((1, F), lambda i: (0, 0))
_p_spec = pl.BlockSpec((NC, _BLK, F), lambda i: (0, i, 0))

_mlp_call = pl.pallas_call(
    _mlp_body,
    grid=(_GRID,),
    in_specs=[_row_spec, _w_spec, _b_spec, _w_spec, _b_spec, _full_deg_spec],
    out_specs=[_row_spec, _row_spec],
    out_shape=[
        jax.ShapeDtypeStruct((NP, F), jnp.float32),
        jax.ShapeDtypeStruct((NP, F), jnp.float32),
    ],
)

_scale_call = pl.pallas_call(
    _scale_body,
    grid=(_GRID,),
    in_specs=[_p_spec, _full_deg_spec],
    out_specs=_row_spec,
    out_shape=jax.ShapeDtypeStruct((NP, F), jnp.float32),
)

_final_call = pl.pallas_call(
    _final_body,
    grid=(_GRID,),
    in_specs=[_p_spec, _full_deg_spec, _row_spec],
    out_specs=_row_spec,
    out_shape=jax.ShapeDtypeStruct((NP, F), jnp.float32),
)


@jax.jit
def kernel(x, edge_index, W1, b1, W2, b2):
  row = edge_index[0]
  col = edge_index[1]
  x_pad = jnp.pad(x, ((0, NP - N), (0, 0)))
  degp = _deg_kernel(row)
  h, g0 = _mlp_call(x_pad, W1, b1.reshape(1, F), W2, b2.reshape(1, F), degp)
  p = _scatter_kernel(g0, row, col)
  g1 = _scale_call(p, degp)
  q = _scatter_kernel(g1, row, col)
  out = _final_call(q, degp, h)
  return out[:N]


# speedup vs baseline: 1.1441x; 1.1441x over previous
import functools

import jax
import jax.numpy as jnp
from jax import lax
from jax.experimental import pallas as pl
from jax.experimental.pallas import tpu as pltpu
from jax.experimental.pallas import tpu_sc as plsc

N = 10000
E = 320000
F = 128
NP = 10240
NC = 2
NS = 16
NW = NC * NS
EPW = E // NW
EB = 80
NB = EPW // EB
GR = 3
NGF = NB // GR
TB = NB % GR
GROWS = GR * EB
RPS = NP // NS
DGR = 5

_mesh = plsc.VectorSubcoreMesh(core_axis_name="c", subcore_axis_name="s")


@functools.partial(
    pl.kernel,
    out_type=jax.ShapeDtypeStruct((NC, NP), jnp.float32),
    mesh=_mesh,
    scratch_types=(
        [
            pltpu.VMEM_SHARED((NP,), jnp.float32),
            pltpu.VMEM((RPS,), jnp.float32),
            pltpu.VMEM((EB,), jnp.float32),
        ]
        + [pltpu.VMEM((EB,), jnp.int32) for _ in range(DGR)]
        + [pltpu.SemaphoreType.DMA for _ in range(2 * DGR)]
    ),
)
def _deg_kernel(row_hbm, degp_hbm, acc, zbuf, onesv, *rest):
  idxv = rest[:DGR]
  isem = rest[DGR:2 * DGR]
  ssem = rest[2 * DGR:3 * DGR]
  c = lax.axis_index("c")
  s = lax.axis_index("s")
  wid = c * NS + s
  base = wid * EPW

  z = jnp.zeros((16,), dtype=jnp.float32)
  for i in range(RPS // 16):
    zbuf[pl.ds(16 * i, 16)] = z
  o = jnp.ones((16,), dtype=jnp.float32)
  for i in range(EB // 16):
    onesv[pl.ds(16 * i, 16)] = o
  pltpu.sync_copy(zbuf, acc.at[pl.ds(RPS * s, RPS)])
  plsc.subcore_barrier()

  def dgroup(g, drain):
    il = []
    for r in range(DGR):
      if drain:
        pltpu.make_async_copy(row_hbm.at[pl.ds(0, EB)], idxv[r],
                              ssem[r]).wait()
      off = base + EB * (g * DGR + r)
      il.append(pltpu.async_copy(row_hbm.at[pl.ds(off, EB)], idxv[r],
                                 isem[r]))
    for r in range(DGR):
      il[r].wait()
      pltpu.async_copy(onesv, acc.at[idxv[r]], ssem[r], add=True)

  dgroup(0, False)

  def body(g, carry):
    dgroup(g, True)
    return carry

  lax.fori_loop(1, NB // DGR, body, 0)
  for r in range(DGR):
    pltpu.make_async_copy(row_hbm.at[pl.ds(0, EB)], idxv[r], ssem[r]).wait()
  plsc.subcore_barrier()
  pltpu.sync_copy(acc.at[pl.ds(RPS * s, RPS)], zbuf)
  pltpu.sync_copy(zbuf, degp_hbm.at[c, pl.ds(RPS * s, RPS)])


@functools.partial(
    pl.kernel,
    out_type=jax.ShapeDtypeStruct((NC, NP, F), jnp.float32),
    mesh=_mesh,
    scratch_types=(
        [
            pltpu.VMEM_SHARED((NP, F), jnp.float32),
            pltpu.VMEM((GROWS, F), jnp.float32),
        ]
        + [pltpu.VMEM((EPW,), jnp.int32)]
        + [pltpu.VMEM((40, F), jnp.float32)]
        + [pltpu.VMEM((EB,), jnp.int32) for _ in range(GR)]
        + [pltpu.SemaphoreType.DMA for _ in range(3 * GR)]
    ),
)
def _scatter_kernel(tab_hbm, row_hbm, col_hbm, out_hbm, acc, rows_v, *rest):
  ridx = rest[0]
  zbuf = rest[1]
  cidx = rest[2:2 + GR]
  sems = rest[2 + GR:]
  cisem = sems[:GR]
  gsem = sems[GR:2 * GR]
  ssem = sems[2 * GR:3 * GR]
  c = lax.axis_index("c")
  s = lax.axis_index("s")
  wid = c * NS + s
  base = wid * EPW

  pltpu.async_copy(row_hbm.at[pl.ds(base, EPW)], ridx, cisem[0]).wait()
  g0gd, g0cl = [], []
  for r in range(GR):
    g0gd.append(pltpu.async_copy(tab_hbm.at[ridx.at[pl.ds(EB * r, EB)]],
                                 rows_v.at[pl.ds(EB * r, EB)], gsem[r]))
    g0cl.append(pltpu.async_copy(col_hbm.at[pl.ds(base + EB * r, EB)],
                                 cidx[r], cisem[r]))

  def zrow(r, carry):
    z = jnp.zeros((16,), dtype=jnp.float32)
    for k in range(F // 16):
      zbuf[r, pl.ds(16 * k, 16)] = z
    return carry

  lax.fori_loop(0, 40, zrow, 0)
  zd = []
  for j in range(RPS // 40):
    zd.append(pltpu.async_copy(zbuf, acc.at[pl.ds(RPS * s + 40 * j, 40)],
                               ssem[j % GR]))
  for d in zd:
    d.wait()
  plsc.subcore_barrier()
  for r in range(GR):
    g0gd[r].wait()
    g0cl[r].wait()
    pltpu.async_copy(rows_v.at[pl.ds(EB * r, EB)], acc.at[cidx[r]],
                     ssem[r], add=True)

  def drain_scatter(r):
    pltpu.make_async_copy(tab_hbm.at[pl.ds(0, EB)],
                          rows_v.at[pl.ds(EB * r, EB)], ssem[r]).wait()

  def group(g, nr, drain):
    cl, gd = [], []
    for r in range(nr):
      if drain:
        drain_scatter(r)
      off = base + EB * (g * GR + r)
      cl.append(pltpu.async_copy(col_hbm.at[pl.ds(off, EB)], cidx[r],
                                 cisem[r]))
      gd.append(
          pltpu.async_copy(tab_hbm.at[ridx.at[pl.ds(EB * (g * GR + r), EB)]],
                           rows_v.at[pl.ds(EB * r, EB)], gsem[r]))
    for r in range(nr):
      gd[r].wait()
      cl[r].wait()
      pltpu.async_copy(rows_v.at[pl.ds(EB * r, EB)], acc.at[cidx[r]],
                       ssem[r], add=True)

  def body(g, carry):
    group(g, GR, True)
    return carry

  lax.fori_loop(1, NGF, body, 0)
  if TB:
    group(NGF, TB, True)
  for r in range(GR):
    drain_scatter(r)
  plsc.subcore_barrier()
  dd = []
  for j in range(RPS // 160):
    off = RPS * s + 160 * j
    dd.append(pltpu.async_copy(acc.at[pl.ds(off, 160)],
                               out_hbm.at[c, pl.ds(off, 160)], gsem[j % GR]))
  for d in dd:
    d.wait()


def _mlp_body(x_ref, w1_ref, b1_ref, w2_ref, b2_ref, degp_ref, h_ref,
              g0_ref):
  i = pl.program_id(0)
  h1 = lax.dot_general(x_ref[...], w1_ref[...], (((1,), (1,)), ((), ())),
                       preferred_element_type=jnp.float32)
  h1 = jnp.maximum(h1 + b1_ref[...], 0.0)
  h = lax.dot_general(h1, w2_ref[...], (((1,), (1,)), ((), ())),
                      preferred_element_type=jnp.float32)
  h = h + b2_ref[...]
  h_ref[...] = h
  dp = degp_ref[:, pl.ds(i * _BLK, _BLK)]
  deg = jnp.sum(dp, axis=0)
  dis = jnp.where(deg > 0, lax.rsqrt(deg), 0.0)
  g0_ref[...] = h * dis[:, None]


def _scale_body(p_ref, degp_ref, g1_ref):
  i = pl.program_id(0)
  ps = p_ref[0] + p_ref[1]
  dp = degp_ref[:, pl.ds(i * _BLK, _BLK)]
  deg = jnp.sum(dp, axis=0)
  dis2 = jnp.where(deg > 0, 1.0 / deg, 0.0)
  g1_ref[...] = ps * dis2[:, None]


def _final_body(q_ref, degp_ref, h_ref, o_ref):
  i = pl.program_id(0)
  qs = q_ref[0] + q_ref[1]
  dp = degp_ref[:, pl.ds(i * _BLK, _BLK)]
  deg = jnp.sum(dp, axis=0)
  dis = jnp.where(deg > 0, lax.rsqrt(deg), 0.0)
  o_ref[...] = 0.1 * h_ref[...] + 0.081 * (qs * dis[:, None])


_BLK = 2048
_GRID = NP // _BLK
_full_deg_spec = pl.BlockSpec((NC, NP), lambda i: (0, 0))
_row_spec = pl.BlockSpec((_BLK, F), lambda i: (i, 0))
_w_spec = pl.BlockSpec((F, F), lambda i: (0, 0))
_b_spec = pl.BlockSpec((1, F), lambda i: (0, 0))
_p_spec = pl.BlockSpec((NC, _BLK, F), lambda i: (0, i, 0))

_mlp_call = pl.pallas_call(
    _mlp_body,
    grid=(_GRID,),
    in_specs=[_row_spec, _w_spec, _b_spec, _w_spec, _b_spec, _full_deg_spec],
    out_specs=[_row_spec, _row_spec],
    out_shape=[
        jax.ShapeDtypeStruct((NP, F), jnp.float32),
        jax.ShapeDtypeStruct((NP, F), jnp.float32),
    ],
)

_scale_call = pl.pallas_call(
    _scale_body,
    grid=(_GRID,),
    in_specs=[_p_spec, _full_deg_spec],
    out_specs=_row_spec,
    out_shape=jax.ShapeDtypeStruct((NP, F), jnp.float32),
)

_final_call = pl.pallas_call(
    _final_body,
    grid=(_GRID,),
    in_specs=[_p_spec, _full_deg_spec, _row_spec],
    out_specs=_row_spec,
    out_shape=jax.ShapeDtypeStruct((NP, F), jnp.float32),
)


@jax.jit
def kernel(x, edge_index, W1, b1, W2, b2):
  row = edge_index[0]
  col = edge_index[1]
  x_pad = jnp.pad(x, ((0, NP - N), (0, 0)))
  degp = _deg_kernel(row)
  h, g0 = _mlp_call(x_pad, W1, b1.reshape(1, F), W2, b2.reshape(1, F), degp)
  p = _scatter_kernel(g0, row, col)
  g1 = _scale_call(p, degp)
  q = _scatter_kernel(g1, row, col)
  out = _final_call(q, degp, h)
  return out[:N]
